# baseline (speedup 1.0000x reference)
import functools

import jax
import jax.numpy as jnp
from jax import lax
from jax.experimental import pallas as pl
from jax.experimental.pallas import tpu as pltpu

_BN_EPS = 1e-5


def _moments_kernel(x_ref, mom_ref):
    x = x_ref[0]
    s = jnp.sum(x, axis=1, keepdims=True)
    sq = jnp.sum(x * x, axis=1, keepdims=True)
    m = jnp.concatenate([s, sq], axis=1)

    @pl.when(pl.program_id(1) == 0)
    def _():
        mom_ref[...] = jnp.zeros_like(mom_ref)

    mom_ref[0] += m


def _image_moments(x3):
    n, c, hw = x3.shape
    n2 = n // 2
    return pl.pallas_call(
        _moments_kernel,
        grid=(2, n2),
        in_specs=[pl.BlockSpec((1, c, hw), lambda ci, j: (ci * n2 + j, 0, 0))],
        out_specs=pl.BlockSpec((1, c, 2), lambda ci, j: (ci, 0, 0)),
        out_shape=jax.ShapeDtypeStruct((2, c, 2), jnp.float32),
        compiler_params=pltpu.CompilerParams(
            dimension_semantics=("parallel", "arbitrary")),
    )(x3)


def _shifted(piece, d, hw):
    if d == 0:
        return piece
    rows = piece.shape[0]
    if d > 0:
        return jnp.concatenate(
            [piece[:, d:], jnp.zeros((rows, d), piece.dtype)], axis=1)
    return jnp.concatenate(
        [jnp.zeros((rows, -d), piece.dtype), piece[:, :hw + d]], axis=1)


def _layer_kernel(*refs, img_w, cout, nparts, last, inv_count):
    x_refs = refs[:nparts]
    mom_refs = refs[nparts:2 * nparts]
    gamma_ref, beta_ref, wmask_ref, w_ref = refs[2 * nparts:2 * nparts + 4]
    if last:
        o_ref = refs[-1]
        mout_ref = None
    else:
        o_ref, mout_ref = refs[-2:]
    hw = x_refs[0].shape[2]

    parts = []
    row = 0
    for ref, mref in zip(x_refs, mom_refs):
        c = ref.shape[1]
        m = mref[0] + mref[1]
        mean = m[:, 0:1] * inv_count
        var = m[:, 1:2] * inv_count - mean * mean
        scale = gamma_ref[row:row + c] * lax.rsqrt(var + _BN_EPS)
        shift = beta_ref[row:row + c] - mean * scale
        parts.append(jnp.maximum(ref[0] * scale + shift, 0.0))
        row += c
    a = parts[0] if nparts == 1 else jnp.concatenate(parts, axis=0)

    z = jnp.dot(w_ref[...], a, preferred_element_type=jnp.float32)

    mask_l = wmask_ref[0:1, :]
    mask_r = wmask_ref[1:2, :]
    y = None
    for kh in range(3):
        for kw in range(3):
            t = kh * 3 + kw
            d = (kh - 1) * img_w + (kw - 1)
            piece = _shifted(z[t * cout:(t + 1) * cout, :], d, hw)
            if kw == 0:
                piece = piece * mask_l
            elif kw == 2:
                piece = piece * mask_r
            y = piece if y is None else y + piece

    if last:
        row = 0
        for ref in x_refs:
            c = ref.shape[1]
            o_ref[0, row:row + c, :] = ref[0]
            row += c
        o_ref[0, row:row + cout, :] = y
    else:
        o_ref[0] = y
        s = jnp.sum(y, axis=1, keepdims=True)
        sq = jnp.sum(y * y, axis=1, keepdims=True)
        my = jnp.concatenate([s, sq], axis=1)

        @pl.when(pl.program_id(1) == 0)
        def _():
            mout_ref[...] = jnp.zeros_like(mout_ref)

        mout_ref[0] += my


def _layer_call(parts, moms, gamma, beta, wmask, w9, img_w, last):
    n, _, hw = parts[0].shape
    n2 = n // 2
    cin = gamma.shape[0]
    cout = w9.shape[0] // 9
    c_total = cin + cout
    kern = functools.partial(_layer_kernel, img_w=img_w, cout=cout,
                             nparts=len(parts), last=last,
                             inv_count=1.0 / float(n * hw))
    part_specs = [
        pl.BlockSpec((1, p.shape[1], hw), lambda ci, j: (ci * n2 + j, 0, 0))
        for p in parts
    ]
    mom_specs = [
        pl.BlockSpec((2, m.shape[1], 2), lambda ci, j: (0, 0, 0))
        for m in moms
    ]
    in_specs = part_specs + mom_specs + [
        pl.BlockSpec((cin, 1), lambda ci, j: (0, 0)),
        pl.BlockSpec((cin, 1), lambda ci, j: (0, 0)),
        pl.BlockSpec((2, hw), lambda ci, j: (0, 0)),
        pl.BlockSpec((9 * cout, cin), lambda ci, j: (0, 0)),
    ]
    if last:
        out_specs = [
            pl.BlockSpec((1, c_total, hw), lambda ci, j: (ci * n2 + j, 0, 0)),
        ]
        out_shape = [jax.ShapeDtypeStruct((n, c_total, hw), jnp.float32)]
    else:
        out_specs = [
            pl.BlockSpec((1, cout, hw), lambda ci, j: (ci * n2 + j, 0, 0)),
            pl.BlockSpec((1, cout, 2), lambda ci, j: (ci, 0, 0)),
        ]
        out_shape = [
            jax.ShapeDtypeStruct((n, cout, hw), jnp.float32),
            jax.ShapeDtypeStruct((2, cout, 2), jnp.float32),
        ]
    out_rows = c_total if last else cout
    flops = 2 * n * hw * 9 * cin * cout
    bytes_accessed = 4 * (n * cin * hw + w9.size + n * out_rows * hw)
    return pl.pallas_call(
        kern,
        grid=(2, n2),
        in_specs=in_specs,
        out_specs=out_specs,
        out_shape=out_shape,
        compiler_params=pltpu.CompilerParams(
            dimension_semantics=("parallel", "arbitrary")),
        cost_estimate=pl.CostEstimate(
            flops=flops, transcendentals=0, bytes_accessed=bytes_accessed),
    )(*parts, *moms, gamma, beta, wmask, w9)


def kernel(x_nchw,
           gamma_0, beta_0, w_0,
           gamma_1, beta_1, w_1,
           gamma_2, beta_2, w_2,
           gamma_3, beta_3, w_3,
           gamma_4, beta_4, w_4,
           gamma_5, beta_5, w_5):
    params = [
        (gamma_0, beta_0, w_0),
        (gamma_1, beta_1, w_1),
        (gamma_2, beta_2, w_2),
        (gamma_3, beta_3, w_3),
        (gamma_4, beta_4, w_4),
        (gamma_5, beta_5, w_5),
    ]
    n, c0, h, iw = x_nchw.shape
    hw = h * iw
    cout = params[0][2].shape[0]
    c_total = c0 + len(params) * cout
    x3 = x_nchw.reshape(n, c0, hw).astype(jnp.float32)

    col = jnp.arange(hw, dtype=jnp.int32) % iw
    wmask = jnp.stack([(col >= 1), (col <= iw - 2)]).astype(jnp.float32)

    parts = [x3]
    moms = [_image_moments(x3)]
    out = None
    nl = len(params)
    for li, (gamma, beta, wgt) in enumerate(params):
        cin = c0 + li * cout
        w9 = jnp.transpose(wgt, (2, 3, 0, 1)).reshape(9 * cout, cin)
        last = li == nl - 1
        res = _layer_call(parts, moms, gamma.reshape(cin, 1),
                          beta.reshape(cin, 1), wmask, w9, iw, last)
        if last:
            out = res[0]
        else:
            parts.append(res[0])
            moms.append(res[1])

    return out.reshape(n, c_total, h, iw)


# speedup vs baseline: 1.0482x; 1.0482x over previous
import functools

import jax
import jax.numpy as jnp
from jax import lax
from jax.experimental import pallas as pl
from jax.experimental.pallas import tpu as pltpu

_BN_EPS = 1e-5


def _prologue_kernel(x_ref, buf_ref, xb_ref, mom_ref):
    x = x_ref[0]
    buf_ref[0] = x
    xb_ref[0] = x.astype(jnp.bfloat16)
    s = jnp.sum(x, axis=1, keepdims=True)
    sq = jnp.sum(x * x, axis=1, keepdims=True)
    m = jnp.concatenate([s, sq], axis=1)

    @pl.when(pl.program_id(1) == 0)
    def _():
        mom_ref[...] = jnp.zeros_like(mom_ref)

    mom_ref[0] += m


def _prologue(x3, c_total):
    n, c, hw = x3.shape
    n2 = n // 2
    return pl.pallas_call(
        _prologue_kernel,
        grid=(2, n2),
        in_specs=[pl.BlockSpec((1, c, hw), lambda ci, j: (ci * n2 + j, 0, 0))],
        out_specs=[
            pl.BlockSpec((1, c, hw), lambda ci, j: (ci * n2 + j, 0, 0)),
            pl.BlockSpec((1, c, hw), lambda ci, j: (ci * n2 + j, 0, 0)),
            pl.BlockSpec((1, c, 2), lambda ci, j: (ci, 0, 0)),
        ],
        out_shape=[
            jax.ShapeDtypeStruct((n, c_total, hw), jnp.float32),
            jax.ShapeDtypeStruct((n, c, hw), jnp.bfloat16),
            jax.ShapeDtypeStruct((2, c, 2), jnp.float32),
        ],
        compiler_params=pltpu.CompilerParams(
            dimension_semantics=("parallel", "arbitrary")),
    )(x3)


def _shifted(piece, d, hw):
    if d == 0:
        return piece
    rows = piece.shape[0]
    if d > 0:
        return jnp.concatenate(
            [piece[:, d:], jnp.zeros((rows, d), piece.dtype)], axis=1)
    return jnp.concatenate(
        [jnp.zeros((rows, -d), piece.dtype), piece[:, :hw + d]], axis=1)


def _layer_kernel(*refs, img_w, cout, nparts, last, inv_count):
    x_refs = refs[1:1 + nparts]
    mom_refs = refs[1 + nparts:1 + 2 * nparts]
    gamma_ref, beta_ref, wmask_ref, w_ref = refs[1 + 2 * nparts:5 + 2 * nparts]
    if last:
        buf_ref = refs[-1]
        yb_ref = mout_ref = None
    else:
        buf_ref, yb_ref, mout_ref = refs[-3:]
    hw = x_refs[0].shape[2]

    parts = []
    row = 0
    for ref, mref in zip(x_refs, mom_refs):
        c = ref.shape[1]
        m = mref[0] + mref[1]
        mean = m[:, 0:1] * inv_count
        var = m[:, 1:2] * inv_count - mean * mean
        scale = gamma_ref[row:row + c] * lax.rsqrt(var + _BN_EPS)
        shift = beta_ref[row:row + c] - mean * scale
        xin = ref[0].astype(jnp.float32)
        parts.append(jnp.maximum(xin * scale + shift, 0.0))
        row += c
    a = parts[0] if nparts == 1 else jnp.concatenate(parts, axis=0)

    z = jnp.dot(w_ref[...], a, preferred_element_type=jnp.float32)

    mask_l = wmask_ref[0:1, :]
    mask_r = wmask_ref[1:2, :]
    y = None
    for kh in range(3):
        for kw in range(3):
            t = kh * 3 + kw
            d = (kh - 1) * img_w + (kw - 1)
            piece = _shifted(z[t * cout:(t + 1) * cout, :], d, hw)
            if kw == 0:
                piece = piece * mask_l
            elif kw == 2:
                piece = piece * mask_r
            y = piece if y is None else y + piece

    buf_ref[0] = y
    if not last:
        yb_ref[0] = y.astype(jnp.bfloat16)
        s = jnp.sum(y, axis=1, keepdims=True)
        sq = jnp.sum(y * y, axis=1, keepdims=True)
        my = jnp.concatenate([s, sq], axis=1)

        @pl.when(pl.program_id(1) == 0)
        def _():
            mout_ref[...] = jnp.zeros_like(mout_ref)

        mout_ref[0] += my


def _layer_call(buf, parts, moms, gamma, beta, wmask, w9, img_w, last):
    n, c_total, hw = buf.shape
    n2 = n // 2
    cin = gamma.shape[0]
    cout = w9.shape[0] // 9
    kern = functools.partial(_layer_kernel, img_w=img_w, cout=cout,
                             nparts=len(parts), last=last,
                             inv_count=1.0 / float(n * hw))
    part_specs = [
        pl.BlockSpec((1, p.shape[1], hw), lambda ci, j: (ci * n2 + j, 0, 0))
        for p in parts
    ]
    mom_specs = [
        pl.BlockSpec((2, m.shape[1], 2), lambda ci, j: (0, 0, 0))
        for m in moms
    ]
    in_specs = [
        pl.BlockSpec((1, 8, 128), lambda ci, j: (0, 0, 0)),
    ] + part_specs + mom_specs + [
        pl.BlockSpec((cin, 1), lambda ci, j: (0, 0)),
        pl.BlockSpec((cin, 1), lambda ci, j: (0, 0)),
        pl.BlockSpec((2, hw), lambda ci, j: (0, 0)),
        pl.BlockSpec((9 * cout, cin), lambda ci, j: (0, 0)),
    ]
    row_block = cin // cout
    out_specs = [
        pl.BlockSpec((1, cout, hw), lambda ci, j: (ci * n2 + j, row_block, 0)),
    ]
    out_shape = [jax.ShapeDtypeStruct((n, c_total, hw), jnp.float32)]
    if not last:
        out_specs += [
            pl.BlockSpec((1, cout, hw), lambda ci, j: (ci * n2 + j, 0, 0)),
            pl.BlockSpec((1, cout, 2), lambda ci, j: (ci, 0, 0)),
        ]
        out_shape += [
            jax.ShapeDtypeStruct((n, cout, hw), jnp.bfloat16),
            jax.ShapeDtypeStruct((2, cout, 2), jnp.float32),
        ]
    flops = 2 * n * hw * 9 * cin * cout
    bytes_accessed = 2 * n * cin * hw + 6 * n * cout * hw + 4 * w9.size
    return pl.pallas_call(
        kern,
        grid=(2, n2),
        in_specs=in_specs,
        out_specs=out_specs,
        out_shape=out_shape,
        input_output_aliases={0: 0},
        compiler_params=pltpu.CompilerParams(
            dimension_semantics=("parallel", "arbitrary")),
        cost_estimate=pl.CostEstimate(
            flops=flops, transcendentals=0, bytes_accessed=bytes_accessed),
    )(buf, *parts, *moms, gamma, beta, wmask, w9)


def kernel(x_nchw,
           gamma_0, beta_0, w_0,
           gamma_1, beta_1, w_1,
           gamma_2, beta_2, w_2,
           gamma_3, beta_3, w_3,
           gamma_4, beta_4, w_4,
           gamma_5, beta_5, w_5):
    params = [
        (gamma_0, beta_0, w_0),
        (gamma_1, beta_1, w_1),
        (gamma_2, beta_2, w_2),
        (gamma_3, beta_3, w_3),
        (gamma_4, beta_4, w_4),
        (gamma_5, beta_5, w_5),
    ]
    n, c0, h, iw = x_nchw.shape
    hw = h * iw
    cout = params[0][2].shape[0]
    c_total = c0 + len(params) * cout
    x3 = x_nchw.reshape(n, c0, hw).astype(jnp.float32)

    col = jnp.arange(hw, dtype=jnp.int32) % iw
    wmask = jnp.stack([(col >= 1), (col <= iw - 2)]).astype(jnp.float32)

    buf, xb, momx = _prologue(x3, c_total)
    parts = [xb]
    moms = [momx]
    nl = len(params)
    for li, (gamma, beta, wgt) in enumerate(params):
        cin = c0 + li * cout
        w9 = jnp.transpose(wgt, (2, 3, 0, 1)).reshape(9 * cout, cin)
        last = li == nl - 1
        res = _layer_call(buf, parts, moms, gamma.reshape(cin, 1),
                          beta.reshape(cin, 1), wmask, w9, iw, last)
        buf = res[0]
        if not last:
            parts.append(res[1])
            moms.append(res[2])

    return buf.reshape(n, c_total, h, iw)


# speedup vs baseline: 2.0755x; 1.9801x over previous
import functools

import jax
import jax.numpy as jnp
from jax import lax
from jax.experimental import pallas as pl
from jax.experimental.pallas import tpu as pltpu

_BN_EPS = 1e-5
_B = 8


def _prologue_kernel(x_ref, buf_ref, xb_ref, mom_ref):
    x = x_ref[...]
    buf_ref[...] = x
    xb_ref[...] = x.astype(jnp.bfloat16)
    s = jnp.sum(jnp.sum(x, axis=2, keepdims=True), axis=0)
    sq = jnp.sum(jnp.sum(x * x, axis=2, keepdims=True), axis=0)
    m = jnp.concatenate([s, sq], axis=1)

    @pl.when(pl.program_id(1) == 0)
    def _():
        mom_ref[...] = jnp.zeros_like(mom_ref)

    mom_ref[0] += m


def _prologue(x3, c_total):
    n, c, hw = x3.shape
    nb = n // (2 * _B)
    return pl.pallas_call(
        _prologue_kernel,
        grid=(2, nb),
        in_specs=[pl.BlockSpec((_B, c, hw),
                               lambda ci, j: (ci * nb + j, 0, 0))],
        out_specs=[
            pl.BlockSpec((_B, c, hw), lambda ci, j: (ci * nb + j, 0, 0)),
            pl.BlockSpec((_B, c, hw), lambda ci, j: (ci * nb + j, 0, 0)),
            pl.BlockSpec((1, c, 2), lambda ci, j: (ci, 0, 0)),
        ],
        out_shape=[
            jax.ShapeDtypeStruct((n, c_total, hw), jnp.float32),
            jax.ShapeDtypeStruct((n, c, hw), jnp.bfloat16),
            jax.ShapeDtypeStruct((2, c, 2), jnp.float32),
        ],
        compiler_params=pltpu.CompilerParams(
            dimension_semantics=("parallel", "arbitrary")),
    )(x3)


def _shifted(piece, d, hw):
    if d == 0:
        return piece
    pad = piece.shape[:-1] + (abs(d),)
    if d > 0:
        return jnp.concatenate(
            [piece[..., d:], jnp.zeros(pad, piece.dtype)], axis=-1)
    return jnp.concatenate(
        [jnp.zeros(pad, piece.dtype), piece[..., :hw + d]], axis=-1)


def _layer_kernel(*refs, img_w, cout, nparts, last, inv_count):
    x_refs = refs[1:1 + nparts]
    mom_refs = refs[1 + nparts:1 + 2 * nparts]
    gamma_ref, beta_ref, wmask_ref, w_ref = refs[1 + 2 * nparts:5 + 2 * nparts]
    if last:
        buf_ref = refs[-1]
        yb_ref = mout_ref = None
    else:
        buf_ref, yb_ref, mout_ref = refs[-3:]
    hw = x_refs[0].shape[2]

    parts = []
    row = 0
    for ref, mref in zip(x_refs, mom_refs):
        c = ref.shape[1]
        m = mref[0] + mref[1]
        mean = m[:, 0:1] * inv_count
        var = m[:, 1:2] * inv_count - mean * mean
        scale = gamma_ref[row:row + c] * lax.rsqrt(var + _BN_EPS)
        shift = beta_ref[row:row + c] - mean * scale
        xin = ref[...].astype(jnp.float32)
        parts.append(jnp.maximum(xin * scale[None] + shift[None], 0.0))
        row += c
    a = parts[0] if nparts == 1 else jnp.concatenate(parts, axis=1)

    z = jnp.stack([
        jnp.dot(w_ref[...], a[b], preferred_element_type=jnp.float32)
        for b in range(a.shape[0])
    ])

    mask_l = wmask_ref[0:1, :]
    mask_r = wmask_ref[1:2, :]
    y = None
    for kh in range(3):
        for kw in range(3):
            t = kh * 3 + kw
            d = (kh - 1) * img_w + (kw - 1)
            piece = _shifted(z[:, t * cout:(t + 1) * cout, :], d, hw)
            if kw == 0:
                piece = piece * mask_l
            elif kw == 2:
                piece = piece * mask_r
            y = piece if y is None else y + piece

    buf_ref[...] = y
    if not last:
        yb_ref[...] = y.astype(jnp.bfloat16)
        s = jnp.sum(jnp.sum(y, axis=2, keepdims=True), axis=0)
        sq = jnp.sum(jnp.sum(y * y, axis=2, keepdims=True), axis=0)
        my = jnp.concatenate([s, sq], axis=1)

        @pl.when(pl.program_id(1) == 0)
        def _():
            mout_ref[...] = jnp.zeros_like(mout_ref)

        mout_ref[0] += my


def _layer_call(buf, parts, moms, gamma, beta, wmask, w9, img_w, last):
    n, c_total, hw = buf.shape
    nb = n // (2 * _B)
    cin = gamma.shape[0]
    cout = w9.shape[0] // 9
    kern = functools.partial(_layer_kernel, img_w=img_w, cout=cout,
                             nparts=len(parts), last=last,
                             inv_count=1.0 / float(n * hw))
    part_specs = [
        pl.BlockSpec((_B, p.shape[1], hw), lambda ci, j: (ci * nb + j, 0, 0))
        for p in parts
    ]
    mom_specs = [
        pl.BlockSpec((2, m.shape[1], 2), lambda ci, j: (0, 0, 0))
        for m in moms
    ]
    in_specs = [
        pl.BlockSpec((1, 8, 128 if hw % 128 == 0 else hw),
                     lambda ci, j: (0, 0, 0)),
    ] + part_specs + mom_specs + [
        pl.BlockSpec((cin, 1), lambda ci, j: (0, 0)),
        pl.BlockSpec((cin, 1), lambda ci, j: (0, 0)),
        pl.BlockSpec((2, hw), lambda ci, j: (0, 0)),
        pl.BlockSpec((9 * cout, cin), lambda ci, j: (0, 0)),
    ]
    row_block = cin // cout
    out_specs = [
        pl.BlockSpec((_B, cout, hw),
                     lambda ci, j: (ci * nb + j, row_block, 0)),
    ]
    out_shape = [jax.ShapeDtypeStruct((n, c_total, hw), jnp.float32)]
    if not last:
        out_specs += [
            pl.BlockSpec((_B, cout, hw), lambda ci, j: (ci * nb + j, 0, 0)),
            pl.BlockSpec((1, cout, 2), lambda ci, j: (ci, 0, 0)),
        ]
        out_shape += [
            jax.ShapeDtypeStruct((n, cout, hw), jnp.bfloat16),
            jax.ShapeDtypeStruct((2, cout, 2), jnp.float32),
        ]
    flops = 2 * n * hw * 9 * cin * cout
    bytes_accessed = 2 * n * cin * hw + 6 * n * cout * hw + 4 * w9.size
    return pl.pallas_call(
        kern,
        grid=(2, nb),
        in_specs=in_specs,
        out_specs=out_specs,
        out_shape=out_shape,
        input_output_aliases={0: 0},
        compiler_params=pltpu.CompilerParams(
            dimension_semantics=("parallel", "arbitrary")),
        cost_estimate=pl.CostEstimate(
            flops=flops, transcendentals=0, bytes_accessed=bytes_accessed),
    )(buf, *parts, *moms, gamma, beta, wmask, w9)


def kernel(x_nchw,
           gamma_0, beta_0, w_0,
           gamma_1, beta_1, w_1,
           gamma_2, beta_2, w_2,
           gamma_3, beta_3, w_3,
           gamma_4, beta_4, w_4,
           gamma_5, beta_5, w_5):
    params = [
        (gamma_0, beta_0, w_0),
        (gamma_1, beta_1, w_1),
        (gamma_2, beta_2, w_2),
        (gamma_3, beta_3, w_3),
        (gamma_4, beta_4, w_4),
        (gamma_5, beta_5, w_5),
    ]
    n, c0, h, iw = x_nchw.shape
    hw = h * iw
    cout = params[0][2].shape[0]
    c_total = c0 + len(params) * cout
    x3 = x_nchw.reshape(n, c0, hw).astype(jnp.float32)

    col = jnp.arange(hw, dtype=jnp.int32) % iw
    wmask = jnp.stack([(col >= 1), (col <= iw - 2)]).astype(jnp.float32)

    buf, xb, momx = _prologue(x3, c_total)
    parts = [xb]
    moms = [momx]
    nl = len(params)
    for li, (gamma, beta, wgt) in enumerate(params):
        cin = c0 + li * cout
        w9 = jnp.transpose(wgt, (2, 3, 0, 1)).reshape(9 * cout, cin)
        last = li == nl - 1
        res = _layer_call(buf, parts, moms, gamma.reshape(cin, 1),
                          beta.reshape(cin, 1), wmask, w9, iw, last)
        buf = res[0]
        if not last:
            parts.append(res[1])
            moms.append(res[2])

    return buf.reshape(n, c_total, h, iw)
